# baseline (speedup 1.0000x reference)
import numpy as np
import jax
import jax.numpy as jnp
from jax import lax
from jax.experimental import pallas as pl
from jax.experimental.pallas import tpu as pltpu
from jax.experimental.pallas import tpu_sc as plsc

_VOCAB = 100000
_D = 64
_B = 4096
_L = 200
_SCALE = 8.0

_NC, _NS, _LANES = 2, 16, 16
_NW = _NC * _NS
_SPW = _B // _NW
_VPR = _D // _LANES
_G0 = 128
_G1 = _L - _G0
_NSUB = _L // _LANES
_TAIL = _L - _NSUB * _LANES


def _pos_encoding():
    depth_per_part = _D // 2
    positions = np.arange(_L)[:, np.newaxis]
    rates = np.arange(depth_per_part)[np.newaxis, :]
    angle_rates = 1 / np.power(10000, 2 * rates / np.float32(_D))
    rads = positions * angle_rates
    return np.concatenate([np.sin(rads), np.cos(rads)], axis=-1).astype(np.float32)


def _body(x_hbm, pe_hbm, table_hbm, out_hbm, idx_v, pe_v, rows_v, sidx_v,
          obuf_v, gsem0, gsem1, osem):
    wid = lax.axis_index("s") * _NC + lax.axis_index("c")
    sbase = wid * _SPW
    pltpu.sync_copy(x_hbm.at[pl.ds(sbase, _SPW)], idx_v)
    pltpu.sync_copy(pe_hbm, pe_v)

    gsems = (gsem0, gsem1)

    def halve(s, b):
        offs = [t * _LANES for t in range(_L // _LANES)] + [_L - _LANES]
        for o in offs:
            sl = pl.ds(o, _LANES)
            sidx_v[b, sl] = lax.shift_right_logical(idx_v[s, sl], 1)

    def fire_gather(s, b):
        pltpu.async_copy(table_hbm.at[sidx_v.at[b, pl.ds(0, _G0)]],
                         rows_v.at[b, pl.ds(0, _G0)], gsems[b])
        pltpu.async_copy(table_hbm.at[sidx_v.at[b, pl.ds(_G0, _G1)]],
                         rows_v.at[b, pl.ds(_G0, _G1)], gsems[b])

    def drain_gather(s, b):
        pltpu.make_async_copy(table_hbm.at[sidx_v.at[b, pl.ds(0, _G0)]],
                              rows_v.at[b, pl.ds(0, _G0)], gsems[b]).wait()
        pltpu.make_async_copy(table_hbm.at[sidx_v.at[b, pl.ds(_G0, _G1)]],
                              rows_v.at[b, pl.ds(_G0, _G1)], gsems[b]).wait()

    def fire_out(s):
        pltpu.async_copy(obuf_v, out_hbm.at[sbase + s], osem)

    def drain_out(s):
        pltpu.make_async_copy(obuf_v, out_hbm.at[sbase + s], osem).wait()

    def compute(s, b):
        buf = rows_v.at[b]

        def do_rows(st, nrows, lane_off=0):
            parvec = lax.rem(
                idx_v[s, pl.ds(st * _LANES - lane_off, _LANES)], 2
            ).astype(jnp.float32)
            for j in range(nrows):
                par = jnp.take(parvec,
                               jnp.full((_LANES,), lane_off + j, jnp.int32))
                i = st * _LANES + j
                ph, pc = lax.shift_right_logical(i, 1), lax.rem(i, 2) * _D
                for t in range(_VPR):
                    sl = pl.ds(t * _LANES, _LANES)
                    hi = buf[i, pl.ds(_D + t * _LANES, _LANES)]
                    lo = buf[i, sl]
                    obuf_v[i, sl] = ((lo + par * (hi - lo)) * _SCALE
                                     + pe_v[ph, pl.ds(pc + t * _LANES,
                                                      _LANES)])

        def sub_fn(st, carry):
            do_rows(st, _LANES)
            return carry

        lax.fori_loop(0, _NSUB, sub_fn, 0)
        do_rows(_NSUB, _TAIL, lane_off=_LANES - _TAIL)

    def phase(s, b, first, fire_next):
        drain_gather(s, b)
        if fire_next:
            halve(s + 1, 1 - b)
            fire_gather(s + 1, 1 - b)
        if not first:
            drain_out(s - 1)
        compute(s, b)
        fire_out(s)

    halve(0, 0)
    fire_gather(0, 0)
    phase(0, 0, True, True)

    def pair(k, carry):
        j = 1 + 2 * k
        phase(j, 1, False, True)
        phase(j + 1, 0, False, True)
        return carry

    lax.fori_loop(0, (_SPW - 2) // 2, pair, 0)

    phase(_SPW - 1, 1, False, False)
    drain_out(_SPW - 1)


def kernel(x, table):
    pe = _pos_encoding().reshape(_L // 2, 2 * _D)
    mesh = plsc.VectorSubcoreMesh(
        core_axis_name="c", subcore_axis_name="s",
        num_cores=_NC, num_subcores=_NS)
    out = pl.kernel(
        _body,
        out_type=jax.ShapeDtypeStruct((_B, _L, _D), jnp.float32),
        mesh=mesh,
        compiler_params=pltpu.CompilerParams(use_tc_tiling_on_sc=True),
        scratch_types=[
            pltpu.VMEM((_SPW, _L), jnp.int32),
            pltpu.VMEM((_L // 2, 2 * _D), jnp.float32),
            pltpu.VMEM((2, _L, 2 * _D), jnp.float32),
            pltpu.VMEM((2, 2 * _G0), jnp.int32),
            pltpu.VMEM((_L, _D), jnp.float32),
            pltpu.SemaphoreType.DMA,
            pltpu.SemaphoreType.DMA,
            pltpu.SemaphoreType.DMA,
        ],
    )(x, jnp.asarray(pe), table.reshape(_VOCAB // 2, 2 * _D))
    return out


# speedup vs baseline: 1.1743x; 1.1743x over previous
import numpy as np
import jax
import jax.numpy as jnp
from jax import lax
from jax.experimental import pallas as pl
from jax.experimental.pallas import tpu as pltpu
from jax.experimental.pallas import tpu_sc as plsc

_VOCAB = 100000
_D = 64
_B = 4096
_L = 200
_SCALE = 8.0

_NC, _NS, _LANES = 2, 16, 16
_NW = _NC * _NS
_ROWS = _B * _L
_RPW = _ROWS // _NW
_CHUNK = 128
_NPHASE = _RPW // _CHUNK
_VPR = _D // _LANES
_PE_FLAT = _L * _VPR * -(-(_L * _VPR + _CHUNK * _VPR) // (_L * _VPR))
_PE_REP = _PE_FLAT // (_L * _VPR)
_PE_ROWS = _PE_FLAT * _LANES // 128


def _pos_encoding():
    depth_per_part = _D // 2
    positions = np.arange(_L)[:, np.newaxis]
    rates = np.arange(depth_per_part)[np.newaxis, :]
    angle_rates = 1 / np.power(10000, 2 * rates / np.float32(_D))
    rads = positions * angle_rates
    return np.concatenate([np.sin(rads), np.cos(rads)], axis=-1).astype(np.float32)


def _body(x_hbm, pe2_hbm, table_hbm, out_hbm, idx_v, pe2_v, rows_v, sidx_v,
          obuf_v, gsem0, gsem1, osem0, osem1):
    wid = lax.axis_index("s") * _NC + lax.axis_index("c")
    base = wid * _RPW
    pltpu.sync_copy(x_hbm.at[wid], idx_v)
    pltpu.sync_copy(pe2_hbm, pe2_v)

    gsems = (gsem0, gsem1)
    osems = (osem0, osem1)

    def halve(s, b):
        for t in range(_CHUNK // _LANES):
            sl = pl.ds(t * _LANES, _LANES)
            sidx_v[b, sl] = lax.shift_right_logical(idx_v[s, sl], 1)

    def fire_gather(s, b):
        pltpu.async_copy(table_hbm.at[sidx_v.at[b]], rows_v.at[b], gsems[b])

    def drain_gather(s, b):
        pltpu.make_async_copy(
            table_hbm.at[sidx_v.at[b]], rows_v.at[b], gsems[b]).wait()

    def fire_out(s, b):
        pltpu.async_copy(
            obuf_v.at[b], out_hbm.at[pl.ds(base + s * _CHUNK, _CHUNK)],
            osems[b])

    def drain_out(s, b):
        pltpu.make_async_copy(
            obuf_v.at[b], out_hbm.at[pl.ds(base + s * _CHUNK, _CHUNK)],
            osems[b]).wait()

    def compute(s, b):
        buf = rows_v.at[b]
        ob = obuf_v.at[b]
        p0 = lax.rem(s * (_CHUNK * _VPR), _L * _VPR)

        def sub_fn(st, carry):
            parvec = lax.rem(idx_v[s, pl.ds(st * _LANES, _LANES)], 2
                             ).astype(jnp.float32)
            pst = p0 + st * (_LANES * _VPR)
            for j in range(_LANES):
                par = jnp.take(parvec, jnp.full((_LANES,), j, jnp.int32))
                i = st * _LANES + j
                p = pst + _VPR * j
                ph, pc = lax.shift_right_logical(p, 3), lax.rem(p, 8) * _LANES
                for t in range(_VPR):
                    sl = pl.ds(t * _LANES, _LANES)
                    hi = buf[i, pl.ds(_D + t * _LANES, _LANES)]
                    lo = buf[i, sl]
                    ob[i, sl] = ((lo + par * (hi - lo)) * _SCALE
                                 + pe2_v[ph, pl.ds(pc + t * _LANES, _LANES)])
            return carry

        lax.fori_loop(0, _CHUNK // _LANES, sub_fn, 0)

    def phase(s, b, first, fire_next):
        drain_gather(s, b)
        if not first:
            drain_out(s - 1, 1 - b)
        if fire_next:
            halve(s + 1, 1 - b)
            fire_gather(s + 1, 1 - b)
        compute(s, b)
        fire_out(s, b)

    halve(0, 0)
    fire_gather(0, 0)
    phase(0, 0, True, True)
    phase(1, 1, False, True)

    def pair(k, carry):
        j = 2 + 2 * k
        phase(j, 0, False, True)
        phase(j + 1, 1, False, True)
        return carry

    lax.fori_loop(0, (_NPHASE - 4) // 2, pair, 0)

    phase(_NPHASE - 2, 0, False, True)
    phase(_NPHASE - 1, 1, False, False)
    drain_out(_NPHASE - 1, 1)


def kernel(x, table):
    pe = _pos_encoding()
    pe2 = np.concatenate([pe] * _PE_REP, axis=0).reshape(_PE_ROWS, 128)
    xr = x.reshape(_NW, _NPHASE, _CHUNK)
    mesh = plsc.VectorSubcoreMesh(
        core_axis_name="c", subcore_axis_name="s",
        num_cores=_NC, num_subcores=_NS)
    out = pl.kernel(
        _body,
        out_type=jax.ShapeDtypeStruct((_ROWS, _D), jnp.float32),
        mesh=mesh,
        compiler_params=pltpu.CompilerParams(use_tc_tiling_on_sc=True),
        scratch_types=[
            pltpu.VMEM((_NPHASE, _CHUNK), jnp.int32),
            pltpu.VMEM((_PE_ROWS, 128), jnp.float32),
            pltpu.VMEM((2, _CHUNK, 2 * _D), jnp.float32),
            pltpu.VMEM((2, _CHUNK), jnp.int32),
            pltpu.VMEM((2, _CHUNK, _D), jnp.float32),
            pltpu.SemaphoreType.DMA,
            pltpu.SemaphoreType.DMA,
            pltpu.SemaphoreType.DMA,
            pltpu.SemaphoreType.DMA,
        ],
    )(xr, jnp.asarray(pe2), table.reshape(_VOCAB // 2, 2 * _D))
    return out.reshape(_B, _L, _D)


# speedup vs baseline: 1.2118x; 1.0319x over previous
import numpy as np
import jax
import jax.numpy as jnp
from jax import lax
from jax.experimental import pallas as pl
from jax.experimental.pallas import tpu as pltpu
from jax.experimental.pallas import tpu_sc as plsc

_VOCAB = 100000
_D = 64
_B = 4096
_L = 200
_SCALE = 8.0

_NC, _NS, _LANES = 2, 16, 16
_NW = _NC * _NS
_ROWS = _B * _L
_RPW = _ROWS // _NW
_CHUNK = 128
_K = 2
_PCHUNK = _K * _CHUNK
_NPHASE = _RPW // _PCHUNK
_NCHUNK = _RPW // _CHUNK
_VPR = _D // _LANES
_PE_VREGS = _L * _VPR * -(-(_L * _VPR + _PCHUNK * _VPR) // (_L * _VPR))
_PE_REP = _PE_VREGS // (_L * _VPR)


def _pos_encoding():
    depth_per_part = _D // 2
    positions = np.arange(_L)[:, np.newaxis]
    rates = np.arange(depth_per_part)[np.newaxis, :]
    angle_rates = 1 / np.power(10000, 2 * rates / np.float32(_D))
    rads = positions * angle_rates
    return np.concatenate([np.sin(rads), np.cos(rads)], axis=-1).astype(np.float32)


def _body(x_hbm, pe2_hbm, table_hbm, out_hbm, idx_v, pe2_v, rows_v, sidx_v,
          gsem0, gsem1, osem0, osem1):
    wid = lax.axis_index("s") * _NC + lax.axis_index("c")
    base = wid * _RPW
    pltpu.sync_copy(x_hbm.at[wid], idx_v)
    pltpu.sync_copy(pe2_hbm, pe2_v)

    gsems = (gsem0, gsem1)
    osems = (osem0, osem1)

    def halve(s, b):
        for k in range(_K):
            for t in range(_CHUNK // _LANES):
                sl = pl.ds(t * _LANES, _LANES)
                sidx_v[b, k, sl] = lax.shift_right_logical(
                    idx_v[s * _K + k, sl], 1)

    def fire_gather(s, b):
        for k in range(_K):
            pltpu.async_copy(
                table_hbm.at[sidx_v.at[b, k]],
                rows_v.at[b, pl.ds(k * _CHUNK, _CHUNK)], gsems[b])

    def drain_gather(s, b):
        for k in range(_K):
            pltpu.make_async_copy(
                table_hbm.at[sidx_v.at[b, k]],
                rows_v.at[b, pl.ds(k * _CHUNK, _CHUNK)], gsems[b]).wait()

    def fire_out(s, b):
        pltpu.async_copy(
            rows_v.at[b], out_hbm.at[pl.ds(base + s * _PCHUNK, _PCHUNK)],
            osems[b])

    def drain_out(s, b):
        pltpu.make_async_copy(
            rows_v.at[b], out_hbm.at[pl.ds(base + s * _PCHUNK, _PCHUNK)],
            osems[b]).wait()

    def compute(s, b):
        buf = rows_v.at[b]
        p0 = lax.rem(s * (_PCHUNK * _VPR), _L * _VPR)

        def sub_fn(st, carry):
            row2d = s * _K + lax.shift_right_logical(st, 3)
            col0 = lax.rem(st, 8) * _LANES
            parvec = lax.rem(idx_v[row2d, pl.ds(col0, _LANES)], 2
                             ).astype(jnp.float32)
            pst = p0 + st * (_LANES * _VPR)
            for j in range(_LANES):
                par = jnp.take(parvec, jnp.full((_LANES,), j, jnp.int32))
                i = st * _LANES + j
                for t in range(_VPR):
                    sl = pl.ds(t * _LANES, _LANES)
                    hi = buf[i, pl.ds(_D + t * _LANES, _LANES)]
                    lo = buf[i, sl]
                    buf[i, sl] = ((lo + par * (hi - lo)) * _SCALE
                                  + pe2_v[pst + _VPR * j + t])
            return carry

        lax.fori_loop(0, _PCHUNK // _LANES, sub_fn, 0)

    def phase(s, b, first, fire_next):
        drain_gather(s, b)
        if not first:
            drain_out(s - 1, 1 - b)
        if fire_next:
            halve(s + 1, 1 - b)
            fire_gather(s + 1, 1 - b)
        compute(s, b)
        fire_out(s, b)

    halve(0, 0)
    fire_gather(0, 0)
    phase(0, 0, True, True)
    phase(1, 1, False, True)

    def pair(k, carry):
        j = 2 + 2 * k
        phase(j, 0, False, True)
        phase(j + 1, 1, False, True)
        return carry

    lax.fori_loop(0, (_NPHASE - 4) // 2, pair, 0)

    phase(_NPHASE - 2, 0, False, True)
    phase(_NPHASE - 1, 1, False, False)
    drain_out(_NPHASE - 1, 1)


def kernel(x, table):
    pe = _pos_encoding()
    pe2 = np.concatenate([pe] * _PE_REP, axis=0).reshape(_PE_VREGS, _LANES)
    xr = x.reshape(_NW, _NCHUNK, _CHUNK)
    mesh = plsc.VectorSubcoreMesh(
        core_axis_name="c", subcore_axis_name="s",
        num_cores=_NC, num_subcores=_NS)
    out = pl.kernel(
        _body,
        out_type=jax.ShapeDtypeStruct((_ROWS, 128), jnp.float32),
        mesh=mesh,
        compiler_params=pltpu.CompilerParams(use_tc_tiling_on_sc=False),
        scratch_types=[
            pltpu.VMEM((_NCHUNK, _CHUNK), jnp.int32),
            pltpu.VMEM((_PE_VREGS, _LANES), jnp.float32),
            pltpu.VMEM((2, _PCHUNK, 128), jnp.float32),
            pltpu.VMEM((2, _K, _CHUNK), jnp.int32),
            pltpu.SemaphoreType.DMA,
            pltpu.SemaphoreType.DMA,
            pltpu.SemaphoreType.DMA,
            pltpu.SemaphoreType.DMA,
        ],
    )(xr, jnp.asarray(pe2), table.reshape(_VOCAB // 2, 2 * _D))
    return out[:, :_D].reshape(_B, _L, _D)
